# baseline (speedup 1.0000x reference)
import dataclasses
import functools

import jax
import jax.numpy as jnp
from jax import lax
from jax.experimental import pallas as pl
from jax.experimental.pallas import tpu as pltpu
from jax.experimental.pallas import tpu_sc as plsc

N_NODES = 10000
N_EDGES = 320000
D = 128
NC = 2
NS = 16
NW = NC * NS
EPW = N_EDGES // NW
C = 80
NCH = EPW // C
SS = 25
NSS = NCH // SS
ZR = 624
TAIL = N_NODES - NS * ZR
LANES = 16

_mesh = plsc.VectorSubcoreMesh(core_axis_name="c", subcore_axis_name="s")

_cp = pltpu.CompilerParams()
if "needs_layout_passes" in pltpu.CompilerParams.__dataclass_fields__:
    _cp = dataclasses.replace(_cp, needs_layout_passes=False)


@functools.partial(
    pl.kernel,
    out_type=jax.ShapeDtypeStruct((NC, N_NODES, D), jnp.float32),
    mesh=_mesh,
    compiler_params=_cp,
    scratch_types=[
        pltpu.VMEM((SS, C), jnp.int32),
        pltpu.VMEM((SS, C), jnp.int32),
        pltpu.VMEM((SS, C), jnp.float32),
        pltpu.VMEM((C, D), jnp.float32),
        pltpu.VMEM((C, D), jnp.float32),
        pltpu.VMEM_SHARED((N_NODES, D), jnp.float32),
        pltpu.SemaphoreType.DMA,
        pltpu.SemaphoreType.DMA,
        pltpu.SemaphoreType.DMA,
        pltpu.SemaphoreType.DMA,
    ],
)
def _sc_aggregate(x_hbm, src_hbm, dst_hbm, val_hbm, out_hbm,
                  src_v, dst_v, val_v, rows0_v, rows1_v, acc,
                  sem0, sem1, sem2, sem3):
    c = lax.axis_index("c")
    s = lax.axis_index("s")
    wid = c * NS + s

    zero16 = jnp.zeros((LANES,), jnp.float32)

    @pl.loop(0, C)
    def _(r):
        for q in range(D // LANES):
            rows0_v[r, pl.ds(q * LANES, LANES)] = zero16

    base = s * ZR

    @pl.loop(0, (ZR // C) * C, step=C)
    def _(r0):
        pltpu.sync_copy(rows0_v, acc.at[pl.ds(base + r0, C)])

    ztail = ZR % C
    if ztail:
        pltpu.sync_copy(rows0_v.at[pl.ds(0, ztail)],
                        acc.at[pl.ds(base + (ZR // C) * C, ztail)])

    @pl.when(s == NS - 1)
    def _():
        pltpu.sync_copy(rows0_v.at[pl.ds(0, TAIL)],
                        acc.at[pl.ds(NS * ZR, TAIL)])

    plsc.subcore_barrier()

    def scale(rows_ref, k):
        @plsc.parallel_loop(0, C // LANES, step=1, unroll=2)
        def _(j):
            v16 = val_v[k, pl.ds(j * LANES, LANES)]
            dnums = lax.GatherDimensionNumbers(
                offset_dims=(), collapsed_slice_dims=(0,),
                start_index_map=(0,))
            for e in range(LANES):
                bidx = jnp.full((LANES, 1), e, jnp.int32)
                vb = lax.gather(v16, bidx, dnums, (1,),
                                mode=lax.GatherScatterMode.PROMISE_IN_BOUNDS)
                r = j * LANES + e
                for q in range(D // LANES):
                    sl = pl.ds(q * LANES, LANES)
                    rows_ref[r, sl] = rows_ref[r, sl] * vb

    @pl.loop(0, NSS)
    def _(g):
        pltpu.sync_copy(src_hbm.at[wid].at[g], src_v)
        pltpu.sync_copy(dst_hbm.at[wid].at[g], dst_v)
        pltpu.sync_copy(val_hbm.at[wid].at[g], val_v)

        @pl.loop(0, SS - 1, step=2)
        def _(k):
            cp0 = pltpu.async_copy(x_hbm.at[src_v.at[k]], rows0_v, sem0)
            cp1 = pltpu.async_copy(x_hbm.at[src_v.at[k + 1]], rows1_v, sem1)
            cp0.wait()
            scale(rows0_v, k)
            sc0 = pltpu.async_copy(rows0_v, acc.at[dst_v.at[k]], sem2,
                                   add=True)
            cp1.wait()
            scale(rows1_v, k + 1)
            sc1 = pltpu.async_copy(rows1_v, acc.at[dst_v.at[k + 1]], sem3,
                                   add=True)
            sc0.wait()
            sc1.wait()

        cpl = pltpu.async_copy(x_hbm.at[src_v.at[SS - 1]], rows0_v, sem0)
        cpl.wait()
        scale(rows0_v, SS - 1)
        pltpu.sync_copy(rows0_v, acc.at[dst_v.at[SS - 1]], add=True)

    plsc.subcore_barrier()
    pltpu.sync_copy(acc.at[pl.ds(base, ZR)],
                    out_hbm.at[c].at[pl.ds(base, ZR)])

    @pl.when(s == NS - 1)
    def _():
        pltpu.sync_copy(acc.at[pl.ds(NS * ZR, TAIL)],
                        out_hbm.at[c].at[pl.ds(NS * ZR, TAIL)])


_BLK = 1000


def _mm_body(p_ref, w_ref, b_ref, o_ref):
    agg = p_ref[0] + p_ref[1]
    o_ref[...] = jnp.dot(agg, w_ref[...],
                         preferred_element_type=jnp.float32,
                         precision=lax.Precision.HIGHEST) + b_ref[...]


def _tc_matmul(partials, weight, bias2d):
    return pl.pallas_call(
        _mm_body,
        grid=(N_NODES // _BLK,),
        in_specs=[
            pl.BlockSpec((NC, _BLK, D), lambda i: (0, i, 0)),
            pl.BlockSpec((D, D), lambda i: (0, 0)),
            pl.BlockSpec((1, D), lambda i: (0, 0)),
        ],
        out_specs=pl.BlockSpec((_BLK, D), lambda i: (i, 0)),
        out_shape=jax.ShapeDtypeStruct((N_NODES, D), jnp.float32),
    )(partials, weight, bias2d)


def kernel(x, edge_index, adj_values, weight, bias):
    ei = edge_index.astype(jnp.int32)
    src = ei[1].reshape(NW, NSS, SS, C)
    dst = ei[0].reshape(NW, NSS, SS, C)
    vals = adj_values.reshape(NW, NSS, SS, C)
    partials = _sc_aggregate(x, src, dst, vals)
    return _tc_matmul(partials, weight, bias.reshape(1, D))


# speedup vs baseline: 1.0415x; 1.0415x over previous
import dataclasses
import functools

import jax
import jax.numpy as jnp
from jax import lax
from jax.experimental import pallas as pl
from jax.experimental.pallas import tpu as pltpu
from jax.experimental.pallas import tpu_sc as plsc

N_NODES = 10000
N_EDGES = 320000
D = 128
NC = 2
NS = 16
NW = NC * NS
EPW = N_EDGES // NW
C = 40
NCH = EPW // C
SS = 25
NSS = NCH // SS
NBUF = 3
ZR = 624
TAIL = N_NODES - NS * ZR
LANES = 16

_mesh = plsc.VectorSubcoreMesh(core_axis_name="c", subcore_axis_name="s")

_cp = pltpu.CompilerParams()
if "needs_layout_passes" in pltpu.CompilerParams.__dataclass_fields__:
    _cp = dataclasses.replace(_cp, needs_layout_passes=False)


@functools.partial(
    pl.kernel,
    out_type=jax.ShapeDtypeStruct((NC, N_NODES, D), jnp.float32),
    mesh=_mesh,
    compiler_params=_cp,
    scratch_types=[
        pltpu.VMEM((SS, C), jnp.int32),
        pltpu.VMEM((SS, C), jnp.int32),
        pltpu.VMEM((SS, C), jnp.float32),
        pltpu.VMEM((C, D), jnp.float32),
        pltpu.VMEM((C, D), jnp.float32),
        pltpu.VMEM((C, D), jnp.float32),
        pltpu.VMEM_SHARED((N_NODES, D), jnp.float32),
        pltpu.SemaphoreType.DMA,
        pltpu.SemaphoreType.DMA,
        pltpu.SemaphoreType.DMA,
        pltpu.SemaphoreType.DMA,
        pltpu.SemaphoreType.DMA,
        pltpu.SemaphoreType.DMA,
    ],
)
def _sc_aggregate(x_hbm, src_hbm, dst_hbm, val_hbm, out_hbm,
                  src_v, dst_v, val_v, rows0_v, rows1_v, rows2_v, acc,
                  g0, g1, g2, s0, s1, s2):
    rows = (rows0_v, rows1_v, rows2_v)
    gsem = (g0, g1, g2)
    ssem = (s0, s1, s2)
    c = lax.axis_index("c")
    s = lax.axis_index("s")
    wid = c * NS + s

    zero16 = jnp.zeros((LANES,), jnp.float32)

    @pl.loop(0, C)
    def _(r):
        for q in range(D // LANES):
            rows0_v[r, pl.ds(q * LANES, LANES)] = zero16

    base = s * ZR

    @pl.loop(0, (ZR // C) * C, step=C)
    def _(r0):
        pltpu.sync_copy(rows0_v, acc.at[pl.ds(base + r0, C)])

    ztail = ZR % C
    if ztail:
        pltpu.sync_copy(rows0_v.at[pl.ds(0, ztail)],
                        acc.at[pl.ds(base + (ZR // C) * C, ztail)])

    @pl.when(s == NS - 1)
    def _():
        pltpu.sync_copy(rows0_v.at[pl.ds(0, TAIL)],
                        acc.at[pl.ds(NS * ZR, TAIL)])

    plsc.subcore_barrier()

    def scale(rows_ref, k):
        @plsc.parallel_loop(0, C, step=1, unroll=2)
        def _(r):
            kk = jnp.full((LANES,), k, jnp.int32)
            rr = jnp.full((LANES,), r, jnp.int32)
            v16 = plsc.load_gather(val_v, [kk, rr])
            for q in range(D // LANES):
                sl = pl.ds(q * LANES, LANES)
                rows_ref[r, sl] = rows_ref[r, sl] * v16

    @pl.loop(0, NSS)
    def _(g):
        pltpu.sync_copy(src_hbm.at[wid].at[g], src_v)
        pltpu.sync_copy(dst_hbm.at[wid].at[g], dst_v)
        pltpu.sync_copy(val_hbm.at[wid].at[g], val_v)

        gh = [None] * SS
        sh = [None] * SS
        gh[0] = pltpu.async_copy(x_hbm.at[src_v.at[0]], rows[0], gsem[0])
        gh[1] = pltpu.async_copy(x_hbm.at[src_v.at[1]], rows[1], gsem[1])
        for k in range(SS):
            b = k % NBUF
            gh[k].wait()
            scale(rows[b], k)
            sh[k] = pltpu.async_copy(rows[b], acc.at[dst_v.at[k]],
                                     ssem[b], add=True)
            if k + 2 < SS:
                nb = (k + 2) % NBUF
                if k >= 1:
                    sh[k - 1].wait()
                gh[k + 2] = pltpu.async_copy(
                    x_hbm.at[src_v.at[k + 2]], rows[nb], gsem[nb])
        sh[SS - 3].wait()
        sh[SS - 2].wait()
        sh[SS - 1].wait()

    plsc.subcore_barrier()
    pltpu.sync_copy(acc.at[pl.ds(base, ZR)],
                    out_hbm.at[c].at[pl.ds(base, ZR)])

    @pl.when(s == NS - 1)
    def _():
        pltpu.sync_copy(acc.at[pl.ds(NS * ZR, TAIL)],
                        out_hbm.at[c].at[pl.ds(NS * ZR, TAIL)])


_BLK = 1000


def _mm_body(p_ref, w_ref, b_ref, o_ref):
    agg = p_ref[0] + p_ref[1]
    o_ref[...] = jnp.dot(agg, w_ref[...],
                         preferred_element_type=jnp.float32,
                         precision=lax.Precision.HIGHEST) + b_ref[...]


def _tc_matmul(partials, weight, bias2d):
    return pl.pallas_call(
        _mm_body,
        grid=(N_NODES // _BLK,),
        in_specs=[
            pl.BlockSpec((NC, _BLK, D), lambda i: (0, i, 0)),
            pl.BlockSpec((D, D), lambda i: (0, 0)),
            pl.BlockSpec((1, D), lambda i: (0, 0)),
        ],
        out_specs=pl.BlockSpec((_BLK, D), lambda i: (i, 0)),
        out_shape=jax.ShapeDtypeStruct((N_NODES, D), jnp.float32),
    )(partials, weight, bias2d)


def kernel(x, edge_index, adj_values, weight, bias):
    ei = edge_index.astype(jnp.int32)
    src = ei[1].reshape(NW, NSS, SS, C)
    dst = ei[0].reshape(NW, NSS, SS, C)
    vals = adj_values.reshape(NW, NSS, SS, C)
    partials = _sc_aggregate(x, src, dst, vals)
    return _tc_matmul(partials, weight, bias.reshape(1, D))


# speedup vs baseline: 1.2391x; 1.1898x over previous
import dataclasses
import functools

import jax
import jax.numpy as jnp
from jax import lax
from jax.experimental import pallas as pl
from jax.experimental.pallas import tpu as pltpu
from jax.experimental.pallas import tpu_sc as plsc

N_NODES = 10000
N_EDGES = 320000
D = 128
NC = 2
NS = 16
NW = NC * NS
EPW = N_EDGES // NW
C = 40
NCH = EPW // C
SS = 25
NSS = NCH // SS
NBUF = 5
ZR = 624
TAIL = N_NODES - NS * ZR
LANES = 16

_mesh = plsc.VectorSubcoreMesh(core_axis_name="c", subcore_axis_name="s")

_cp = pltpu.CompilerParams()
if "needs_layout_passes" in pltpu.CompilerParams.__dataclass_fields__:
    _cp = dataclasses.replace(_cp, needs_layout_passes=False)


@functools.partial(
    pl.kernel,
    out_type=jax.ShapeDtypeStruct((NC, N_NODES, D), jnp.float32),
    mesh=_mesh,
    compiler_params=_cp,
    scratch_types=[
        pltpu.VMEM((SS, C), jnp.int32),
        pltpu.VMEM((SS, C), jnp.int32),
        pltpu.VMEM((SS, C), jnp.float32),
        *[pltpu.VMEM((C, D), jnp.float32) for _ in range(NBUF)],
        pltpu.VMEM_SHARED((N_NODES, D), jnp.float32),
        *[pltpu.SemaphoreType.DMA for _ in range(2 * NBUF)],
    ],
)
def _sc_aggregate(x_hbm, src_hbm, dst_hbm, val_hbm, out_hbm,
                  src_v, dst_v, val_v, *rest):
    rows = rest[:NBUF]
    acc = rest[NBUF]
    gsem = rest[NBUF + 1:2 * NBUF + 1]
    ssem = rest[2 * NBUF + 1:]
    rows0_v = rows[0]
    c = lax.axis_index("c")
    s = lax.axis_index("s")
    wid = c * NS + s

    zero16 = jnp.zeros((LANES,), jnp.float32)

    @pl.loop(0, C)
    def _(r):
        for q in range(D // LANES):
            rows0_v[r, pl.ds(q * LANES, LANES)] = zero16

    base = s * ZR

    @pl.loop(0, (ZR // C) * C, step=C)
    def _(r0):
        pltpu.sync_copy(rows0_v, acc.at[pl.ds(base + r0, C)])

    ztail = ZR % C
    if ztail:
        pltpu.sync_copy(rows0_v.at[pl.ds(0, ztail)],
                        acc.at[pl.ds(base + (ZR // C) * C, ztail)])

    @pl.when(s == NS - 1)
    def _():
        pltpu.sync_copy(rows0_v.at[pl.ds(0, TAIL)],
                        acc.at[pl.ds(NS * ZR, TAIL)])

    plsc.subcore_barrier()

    def scale(rows_ref, k):
        @plsc.parallel_loop(0, C, step=1, unroll=2)
        def _(r):
            kk = jnp.full((LANES,), k, jnp.int32)
            rr = jnp.full((LANES,), r, jnp.int32)
            v16 = plsc.load_gather(val_v, [kk, rr])
            for q in range(D // LANES):
                sl = pl.ds(q * LANES, LANES)
                rows_ref[r, sl] = rows_ref[r, sl] * v16

    @pl.loop(0, NSS)
    def _(g):
        pltpu.sync_copy(src_hbm.at[wid].at[g], src_v)
        pltpu.sync_copy(dst_hbm.at[wid].at[g], dst_v)
        pltpu.sync_copy(val_hbm.at[wid].at[g], val_v)

        gh = [None] * SS
        sh = [None] * SS
        for k in range(NBUF - 1):
            gh[k] = pltpu.async_copy(x_hbm.at[src_v.at[k]], rows[k],
                                     gsem[k])
        for k in range(SS):
            b = k % NBUF
            gh[k].wait()
            scale(rows[b], k)
            sh[k] = pltpu.async_copy(rows[b], acc.at[dst_v.at[k]],
                                     ssem[b], add=True)
            if k + NBUF - 1 < SS:
                nb = (k + NBUF - 1) % NBUF
                if k >= 1:
                    sh[k - 1].wait()
                gh[k + NBUF - 1] = pltpu.async_copy(
                    x_hbm.at[src_v.at[k + NBUF - 1]], rows[nb], gsem[nb])
        for k in range(max(0, SS - NBUF), SS):
            sh[k].wait()

    plsc.subcore_barrier()
    pltpu.sync_copy(acc.at[pl.ds(base, ZR)],
                    out_hbm.at[c].at[pl.ds(base, ZR)])

    @pl.when(s == NS - 1)
    def _():
        pltpu.sync_copy(acc.at[pl.ds(NS * ZR, TAIL)],
                        out_hbm.at[c].at[pl.ds(NS * ZR, TAIL)])


_BLK = 1000


def _mm_body(p_ref, w_ref, b_ref, o_ref):
    agg = p_ref[0] + p_ref[1]
    o_ref[...] = jnp.dot(agg, w_ref[...],
                         preferred_element_type=jnp.float32,
                         precision=lax.Precision.HIGHEST) + b_ref[...]


def _tc_matmul(partials, weight, bias2d):
    return pl.pallas_call(
        _mm_body,
        grid=(N_NODES // _BLK,),
        in_specs=[
            pl.BlockSpec((NC, _BLK, D), lambda i: (0, i, 0)),
            pl.BlockSpec((D, D), lambda i: (0, 0)),
            pl.BlockSpec((1, D), lambda i: (0, 0)),
        ],
        out_specs=pl.BlockSpec((_BLK, D), lambda i: (i, 0)),
        out_shape=jax.ShapeDtypeStruct((N_NODES, D), jnp.float32),
    )(partials, weight, bias2d)


def kernel(x, edge_index, adj_values, weight, bias):
    ei = edge_index.astype(jnp.int32)
    src = ei[1].reshape(NW, NSS, SS, C)
    dst = ei[0].reshape(NW, NSS, SS, C)
    vals = adj_values.reshape(NW, NSS, SS, C)
    partials = _sc_aggregate(x, src, dst, vals)
    return _tc_matmul(partials, weight, bias.reshape(1, D))


# speedup vs baseline: 1.3174x; 1.0632x over previous
import dataclasses
import functools

import jax
import jax.numpy as jnp
from jax import lax
from jax.experimental import pallas as pl
from jax.experimental.pallas import tpu as pltpu
from jax.experimental.pallas import tpu_sc as plsc

N_NODES = 10000
N_EDGES = 320000
D = 128
NC = 2
NS = 16
NW = NC * NS
EPW = N_EDGES // NW
C = 40
NCH = EPW // C
SS = 50
NSS = NCH // SS
NBUF = 5
ZR = 624
TAIL = N_NODES - NS * ZR
LANES = 16

_mesh = plsc.VectorSubcoreMesh(core_axis_name="c", subcore_axis_name="s")

_cp = pltpu.CompilerParams()
if "needs_layout_passes" in pltpu.CompilerParams.__dataclass_fields__:
    _cp = dataclasses.replace(_cp, needs_layout_passes=False)


@functools.partial(
    pl.kernel,
    out_type=jax.ShapeDtypeStruct((NC, N_NODES, D), jnp.float32),
    mesh=_mesh,
    compiler_params=_cp,
    scratch_types=[
        pltpu.VMEM((SS, C), jnp.int32),
        pltpu.VMEM((SS, C), jnp.int32),
        pltpu.VMEM((SS, C), jnp.float32),
        *[pltpu.VMEM((C, D), jnp.float32) for _ in range(NBUF)],
        pltpu.VMEM_SHARED((N_NODES, D), jnp.float32),
        *[pltpu.SemaphoreType.DMA for _ in range(2 * NBUF)],
    ],
)
def _sc_aggregate(x_hbm, src_hbm, dst_hbm, val_hbm, out_hbm,
                  src_v, dst_v, val_v, *rest):
    rows = rest[:NBUF]
    acc = rest[NBUF]
    gsem = rest[NBUF + 1:2 * NBUF + 1]
    ssem = rest[2 * NBUF + 1:]
    rows0_v = rows[0]
    c = lax.axis_index("c")
    s = lax.axis_index("s")
    wid = c * NS + s

    zero16 = jnp.zeros((LANES,), jnp.float32)

    @pl.loop(0, C)
    def _(r):
        for q in range(D // LANES):
            rows0_v[r, pl.ds(q * LANES, LANES)] = zero16

    base = s * ZR

    @pl.loop(0, (ZR // C) * C, step=C)
    def _(r0):
        pltpu.sync_copy(rows0_v, acc.at[pl.ds(base + r0, C)])

    ztail = ZR % C
    if ztail:
        pltpu.sync_copy(rows0_v.at[pl.ds(0, ztail)],
                        acc.at[pl.ds(base + (ZR // C) * C, ztail)])

    @pl.when(s == NS - 1)
    def _():
        pltpu.sync_copy(rows0_v.at[pl.ds(0, TAIL)],
                        acc.at[pl.ds(NS * ZR, TAIL)])

    plsc.subcore_barrier()

    def scale(rows_ref, k):
        @plsc.parallel_loop(0, C, step=1, unroll=2)
        def _(r):
            kk = jnp.full((LANES,), k, jnp.int32)
            rr = jnp.full((LANES,), r, jnp.int32)
            v16 = plsc.load_gather(val_v, [kk, rr])
            for q in range(D // LANES):
                sl = pl.ds(q * LANES, LANES)
                rows_ref[r, sl] = rows_ref[r, sl] * v16

    @pl.loop(0, NSS)
    def _(g):
        pltpu.sync_copy(src_hbm.at[wid].at[g], src_v)
        pltpu.sync_copy(dst_hbm.at[wid].at[g], dst_v)
        pltpu.sync_copy(val_hbm.at[wid].at[g], val_v)

        gh = [None] * SS
        sh = [None] * SS
        for k in range(NBUF - 1):
            gh[k] = pltpu.async_copy(x_hbm.at[src_v.at[k]], rows[k],
                                     gsem[k])
        for k in range(SS):
            b = k % NBUF
            gh[k].wait()
            scale(rows[b], k)
            sh[k] = pltpu.async_copy(rows[b], acc.at[dst_v.at[k]],
                                     ssem[b], add=True)
            if k + NBUF - 1 < SS:
                nb = (k + NBUF - 1) % NBUF
                if k >= 1:
                    sh[k - 1].wait()
                gh[k + NBUF - 1] = pltpu.async_copy(
                    x_hbm.at[src_v.at[k + NBUF - 1]], rows[nb], gsem[nb])
        for k in range(max(0, SS - NBUF), SS):
            sh[k].wait()

    plsc.subcore_barrier()
    pltpu.sync_copy(acc.at[pl.ds(base, ZR)],
                    out_hbm.at[c].at[pl.ds(base, ZR)])

    @pl.when(s == NS - 1)
    def _():
        pltpu.sync_copy(acc.at[pl.ds(NS * ZR, TAIL)],
                        out_hbm.at[c].at[pl.ds(NS * ZR, TAIL)])


_BLK = 1000


def _mm_body(p_ref, w_ref, b_ref, o_ref):
    agg = p_ref[0] + p_ref[1]
    o_ref[...] = jnp.dot(agg, w_ref[...],
                         preferred_element_type=jnp.float32,
                         precision=lax.Precision.HIGHEST) + b_ref[...]


def _tc_matmul(partials, weight, bias2d):
    return pl.pallas_call(
        _mm_body,
        grid=(N_NODES // _BLK,),
        in_specs=[
            pl.BlockSpec((NC, _BLK, D), lambda i: (0, i, 0)),
            pl.BlockSpec((D, D), lambda i: (0, 0)),
            pl.BlockSpec((1, D), lambda i: (0, 0)),
        ],
        out_specs=pl.BlockSpec((_BLK, D), lambda i: (i, 0)),
        out_shape=jax.ShapeDtypeStruct((N_NODES, D), jnp.float32),
    )(partials, weight, bias2d)


def kernel(x, edge_index, adj_values, weight, bias):
    ei = edge_index.astype(jnp.int32)
    src = ei[1].reshape(NW, NSS, SS, C)
    dst = ei[0].reshape(NW, NSS, SS, C)
    vals = adj_values.reshape(NW, NSS, SS, C)
    partials = _sc_aggregate(x, src, dst, vals)
    return _tc_matmul(partials, weight, bias.reshape(1, D))
